# baseline (speedup 1.0000x reference)
import functools

import jax
import jax.numpy as jnp
from jax import lax
from jax.experimental import pallas as pl
from jax.experimental.pallas import tpu as pltpu
from jax.experimental.pallas import tpu_sc as plsc

N = 10000
E = 320000
DIN = 126
DH = 128
G = 256
BN_EPS = 1e-5

_GRP = 128
_EP = 327680
_GPT = _EP // (16 * _GRP)
_KI = 8
_NB = _GPT // _KI
_RING = 4
_HALF = N // 2
_NG = _EP // _GRP
_NPAD = 5120
_DUMMY = 5056
_ZR = 320


def _agg_body(h_hbm, srcg_hbm, dstg_hbm, zeros_hbm, out_hbm,
              src_v, dst_v, rows_v, acc,
              is_s, is_d, g0, g1, g2, g3, t0, t1, t2, t3):
    gsem = (g0, g1, g2, g3)
    ssem = (t0, t1, t2, t3)
    cid = lax.axis_index("c")
    sid = lax.axis_index("s")
    lo = cid * _HALF

    def idx_start(b, parity):
        base = sid * _GPT + b * _KI
        pltpu.make_async_copy(srcg_hbm.at[pl.ds(_NG * cid + base, _KI)],
                              src_v.at[parity], is_s).start()
        pltpu.make_async_copy(dstg_hbm.at[pl.ds(_NG * cid + base, _KI)],
                              dst_v.at[parity], is_d).start()

    def idx_wait(b, parity):
        base = sid * _GPT + b * _KI
        pltpu.make_async_copy(srcg_hbm.at[pl.ds(_NG * cid + base, _KI)],
                              src_v.at[parity], is_s).wait()
        pltpu.make_async_copy(dstg_hbm.at[pl.ds(_NG * cid + base, _KI)],
                              dst_v.at[parity], is_d).wait()

    def gather(i, parity, slot):
        return pltpu.make_async_copy(h_hbm.at[src_v.at[parity, i]],
                                     rows_v.at[pl.ds(slot * _GRP, _GRP)],
                                     gsem[slot])

    def scatter(i, parity, slot):
        return pltpu.make_async_copy(rows_v.at[pl.ds(slot * _GRP, _GRP)],
                                     acc.at[dst_v.at[parity, i]], ssem[slot])

    idx_start(0, 0)
    pltpu.sync_copy(zeros_hbm, rows_v.at[pl.ds(0, _ZR)])
    pltpu.sync_copy(rows_v.at[pl.ds(0, _ZR)], acc.at[pl.ds(sid * _ZR, _ZR)])
    plsc.subcore_barrier()

    def body(b, carry):
        pb = b % 2
        idx_wait(b, pb)

        @pl.when(b > 0)
        def _():
            for slot in range(_RING):
                scatter(slot + _KI - _RING, 1 - pb, slot).wait()

        @pl.when(b < _NB - 1)
        def _():
            idx_start(b + 1, 1 - pb)

        for i in range(_KI):
            slot = i % _RING
            if i >= _RING:
                scatter(i - _RING, pb, slot).wait()
            gather(i, pb, slot).start()
            if i >= 1:
                pslot = (i - 1) % _RING
                gather(i - 1, pb, pslot).wait()
                scatter(i - 1, pb, pslot).start(add=True)
        gather(_KI - 1, pb, (_KI - 1) % _RING).wait()
        scatter(_KI - 1, pb, (_KI - 1) % _RING).start(add=True)
        return carry

    lax.fori_loop(0, _NB, body, 0)
    for slot in range(_RING):
        scatter(slot + _KI - _RING, (_NB - 1) % 2, slot).wait()
    plsc.subcore_barrier()

    wb = sid * _ZR
    pltpu.sync_copy(acc.at[pl.ds(wb, _ZR)], rows_v.at[pl.ds(0, _ZR)])

    @pl.when(sid < 15)
    def _():
        pltpu.sync_copy(rows_v.at[pl.ds(0, _ZR)],
                        out_hbm.at[pl.ds(lo + wb, _ZR)])

    @pl.when(sid == 15)
    def _():
        pltpu.sync_copy(rows_v.at[pl.ds(0, 200)],
                        out_hbm.at[pl.ds(lo + wb, 200)])


@functools.lru_cache(maxsize=1)
def _agg_call():
    mesh = plsc.VectorSubcoreMesh(core_axis_name="c", subcore_axis_name="s")
    return pl.kernel(
        _agg_body,
        out_type=jax.ShapeDtypeStruct((N, DH), jnp.float32),
        mesh=mesh,
        scratch_types=[
            pltpu.VMEM((2, _KI, _GRP), jnp.int32),
            pltpu.VMEM((2, _KI, _GRP), jnp.int32),
            pltpu.VMEM((_RING * _GRP, DH), jnp.float32),
            pltpu.VMEM_SHARED((_NPAD, DH), jnp.float32),
        ] + [pltpu.SemaphoreType.DMA] * 10,
    )


def _sc_agg(h, srcg, dstg, zeros):
    return _agg_call()(h, srcg, dstg, zeros)


def _mlp_body(h_ref, agg_ref, w1_ref, b1_ref, g_ref, be_ref, w2_ref,
              b2_ref, out_ref):
    h = h_ref[...] + agg_ref[...]
    y = lax.dot_general(h, w1_ref[...], (((1,), (0,)), ((), ())),
                        preferred_element_type=jnp.float32)
    y = y + b1_ref[...]
    m = jnp.mean(y, axis=0, keepdims=True)
    d = y - m
    v = jnp.mean(d * d, axis=0, keepdims=True)
    y = g_ref[...] * d * lax.rsqrt(v + BN_EPS) + be_ref[...]
    y = jnp.maximum(y, 0.0)
    y = lax.dot_general(y, w2_ref[...], (((1,), (0,)), ((), ())),
                        preferred_element_type=jnp.float32)
    y = y + b2_ref[...]
    out_ref[...] = jnp.maximum(y, 0.0)


def _mlp_call(h, agg, w1, b1, g, be, w2, b2):
    return pl.pallas_call(
        _mlp_body,
        out_shape=jax.ShapeDtypeStruct((N, DH), jnp.float32),
    )(h, agg, w1, b1.reshape(1, -1), g.reshape(1, -1), be.reshape(1, -1),
      w2, b2.reshape(1, -1))


def _tail_body(h_ref, agg_ref, w1_ref, b1_ref, g_ref, be_ref, w2_ref,
               b2_ref, batch_ref, lw1_ref, lb1_ref, lw2_ref, lb2_ref, out_ref):
    h = h_ref[...] + agg_ref[...]
    y = lax.dot_general(h, w1_ref[...], (((1,), (0,)), ((), ())),
                        preferred_element_type=jnp.float32)
    y = y + b1_ref[...]
    m = jnp.mean(y, axis=0, keepdims=True)
    d = y - m
    v = jnp.mean(d * d, axis=0, keepdims=True)
    y = g_ref[...] * d * lax.rsqrt(v + BN_EPS) + be_ref[...]
    y = jnp.maximum(y, 0.0)
    y = lax.dot_general(y, w2_ref[...], (((1,), (0,)), ((), ())),
                        preferred_element_type=jnp.float32)
    y = jnp.maximum(y + b2_ref[...], 0.0)
    seg_ids = batch_ref[...]
    cols = lax.broadcasted_iota(jnp.int32, (N, G), 1)
    onehot = jnp.where(cols == seg_ids, 1.0, 0.0).astype(jnp.float32)
    pooled = lax.dot_general(onehot, y, (((0,), (0,)), ((), ())),
                             preferred_element_type=jnp.float32)
    z = lax.dot_general(pooled, lw1_ref[...], (((1,), (0,)), ((), ())),
                        preferred_element_type=jnp.float32) + lb1_ref[...]
    z = jnp.maximum(z, 0.0)
    z = lax.dot_general(z, lw2_ref[...], (((1,), (0,)), ((), ())),
                        preferred_element_type=jnp.float32) + lb2_ref[...]
    out_ref[...] = 1.0 / (1.0 + jnp.exp(-z))


def _tail_call(h, agg, w1, b1, g, be, w2, b2, batch, lw1, lb1, lw2, lb2):
    return pl.pallas_call(
        _tail_body,
        out_shape=jax.ShapeDtypeStruct((G, 1), jnp.float32),
    )(h, agg, w1, b1.reshape(1, -1), g.reshape(1, -1), be.reshape(1, -1),
      w2, b2.reshape(1, -1), batch.reshape(N, 1), lw1,
      lb1.reshape(1, -1), lw2, lb2.reshape(1, -1))


def kernel(x, edge_index, batch, W11, b11, g1, be1, W12, b12, W21, b21, g2,
           be2, W22, b22, W31, b31, g3, be3, W32, b32, lw1, lb1, lw2, lb2):
    src = edge_index[0]
    dst = edge_index[1]
    pad = _EP - E
    src_p = jnp.concatenate([src, jnp.zeros((pad,), jnp.int32)])
    dst_p = jnp.concatenate([dst, jnp.full((pad,), N, jnp.int32)])
    own0 = dst_p < _HALF
    own1 = (dst_p >= _HALF) & (dst_p < N)
    ldst0 = jnp.where(own0, dst_p, _DUMMY)
    ldst1 = jnp.where(own1, dst_p - _HALF, _DUMMY)
    dstg = jnp.concatenate([ldst0, ldst1]).reshape(-1, _GRP)
    srcg = jnp.concatenate([jnp.where(own0, src_p, 0),
                            jnp.where(own1, src_p, 0)]).reshape(-1, _GRP)
    zeros = jnp.zeros((_ZR, DH), jnp.float32)

    x_pad = jnp.pad(x, ((0, 0), (0, DH - DIN)))
    W11p = jnp.pad(W11, ((0, DH - DIN), (0, 0)))

    agg = _sc_agg(x_pad, srcg, dstg, zeros)
    h = _mlp_call(x_pad, agg, W11p, b11, g1, be1, W12, b12)
    agg = _sc_agg(h, srcg, dstg, zeros)
    h = _mlp_call(h, agg, W21, b21, g2, be2, W22, b22)
    agg = _sc_agg(h, srcg, dstg, zeros)
    return _tail_call(h, agg, W31, b31, g3, be3, W32, b32,
                      batch, lw1, lb1, lw2, lb2)


# speedup vs baseline: 78.1705x; 78.1705x over previous
import functools

import jax
import jax.numpy as jnp
from jax import lax
from jax.experimental import pallas as pl
from jax.experimental.pallas import tpu as pltpu
from jax.experimental.pallas import tpu_sc as plsc

N = 10000
E = 320000
DIN = 126
DH = 128
G = 256
BN_EPS = 1e-5

_GRP = 128
_EP = 327680
_GPT = _EP // (16 * _GRP)
_KI = 8
_NB = _GPT // _KI
_RING = 4
_HALF = N // 2
_NG = _EP // _GRP
_NPAD = 5120
_DUMMY = 5056
_ZR = 320
_PADG = _KI * _GRP
_CAP = _GPT * _GRP + _PADG
_CROWS = _CAP // _GRP
_NT = 32


def _filter_body(srcg_hbm, dstg_hbm, csrc_out, cdst_out, counts_out,
                 src_v, dst_v, csrc, cdst, cnt_v, is_s, is_d):
    cid = lax.axis_index("c")
    sid = lax.axis_index("s")
    tid = cid * 16 + sid

    def idx_start(b, parity):
        base = sid * _GPT + b * _KI
        pltpu.make_async_copy(srcg_hbm.at[pl.ds(base, _KI)],
                              src_v.at[parity], is_s).start()
        pltpu.make_async_copy(dstg_hbm.at[pl.ds(_NG * cid + base, _KI)],
                              dst_v.at[parity], is_d).start()

    def idx_wait(b, parity):
        base = sid * _GPT + b * _KI
        pltpu.make_async_copy(srcg_hbm.at[pl.ds(base, _KI)],
                              src_v.at[parity], is_s).wait()
        pltpu.make_async_copy(dstg_hbm.at[pl.ds(_NG * cid + base, _KI)],
                              dst_v.at[parity], is_d).wait()

    idx_start(0, 0)
    ones = jnp.full((16,), 1, jnp.int32)
    zeros = jnp.full((16,), 0, jnp.int32)
    trash = jnp.full((16,), _CAP, jnp.int32)

    def fbody(bb, cnt):
        for pb in range(2):
            b = bb * 2 + pb
            idx_wait(b, pb)

            @pl.when(b < _NB - 1)
            def _(b=b, pb=pb):
                idx_start(b + 1, 1 - pb)

            for g in range(_KI):
                for k in range(_GRP // 16):
                    s16 = src_v[pb, g, pl.ds(k * 16, 16)]
                    d16 = dst_v[pb, g, pl.ds(k * 16, 16)]
                    m = d16 != _DUMMY
                    mi = jnp.where(m, ones, zeros)
                    pc = plsc.cumsum(mi)
                    pos = jnp.where(m, cnt + pc - 1, trash)
                    plsc.store_scatter(csrc, [pos], s16)
                    plsc.store_scatter(cdst, [pos], d16)
                    cnt = cnt + jnp.sum(mi)
        return cnt

    cnt = lax.fori_loop(0, _NB // 2, fbody, jnp.int32(0))

    iot = lax.iota(jnp.int32, 16)
    dumv = jnp.full((16,), _DUMMY, jnp.int32)
    for k in range(_PADG // 16):
        csrc[pl.ds(cnt + k * 16, 16)] = iot + (k * 16)
        cdst[pl.ds(cnt + k * 16, 16)] = dumv

    nbatch = (cnt + _PADG - 1) // _PADG
    cnt_v[...] = zeros + nbatch

    ob = tid * _CAP
    pltpu.sync_copy(csrc.at[pl.ds(0, _CAP)], csrc_out.at[pl.ds(ob, _CAP)])
    pltpu.sync_copy(cdst.at[pl.ds(0, _CAP)], cdst_out.at[pl.ds(ob, _CAP)])
    pltpu.sync_copy(cnt_v, counts_out.at[pl.ds(tid * 16, 16)])


@functools.lru_cache(maxsize=1)
def _filter_call():
    mesh = plsc.VectorSubcoreMesh(core_axis_name="c", subcore_axis_name="s")
    return pl.kernel(
        _filter_body,
        out_type=[
            jax.ShapeDtypeStruct((_NT * _CAP,), jnp.int32),
            jax.ShapeDtypeStruct((_NT * _CAP,), jnp.int32),
            jax.ShapeDtypeStruct((_NT * 16,), jnp.int32),
        ],
        mesh=mesh,
        compiler_params=pltpu.CompilerParams(needs_layout_passes=False),
        scratch_types=[
            pltpu.VMEM((2, _KI, _GRP), jnp.int32),
            pltpu.VMEM((2, _KI, _GRP), jnp.int32),
            pltpu.VMEM((_CAP + 16,), jnp.int32),
            pltpu.VMEM((_CAP + 16,), jnp.int32),
            pltpu.VMEM((16,), jnp.int32),
            pltpu.SemaphoreType.DMA,
            pltpu.SemaphoreType.DMA,
        ],
    )


def _stream_body(h_hbm, csrc_hbm, cdst_hbm, counts_hbm, zeros_hbm, out_hbm,
                 idxs, idxd, cnt_v, rows_v, acc,
                 is_s, is_d, g0, g1, g2, g3, t0, t1, t2, t3):
    gsem = (g0, g1, g2, g3)
    ssem = (t0, t1, t2, t3)
    cid = lax.axis_index("c")
    sid = lax.axis_index("s")
    tid = cid * 16 + sid
    lo = cid * _HALF

    def idx_start(b, parity):
        base = tid * _CROWS + b * _KI
        pltpu.make_async_copy(csrc_hbm.at[pl.ds(base, _KI)],
                              idxs.at[parity], is_s).start()
        pltpu.make_async_copy(cdst_hbm.at[pl.ds(base, _KI)],
                              idxd.at[parity], is_d).start()

    def idx_wait(b, parity):
        base = tid * _CROWS + b * _KI
        pltpu.make_async_copy(csrc_hbm.at[pl.ds(base, _KI)],
                              idxs.at[parity], is_s).wait()
        pltpu.make_async_copy(cdst_hbm.at[pl.ds(base, _KI)],
                              idxd.at[parity], is_d).wait()

    def gather(i, parity, slot):
        return pltpu.make_async_copy(h_hbm.at[idxs.at[parity, i]],
                                     rows_v.at[pl.ds(slot * _GRP, _GRP)],
                                     gsem[slot])

    def scatter(i, parity, slot):
        return pltpu.make_async_copy(rows_v.at[pl.ds(slot * _GRP, _GRP)],
                                     acc.at[idxd.at[parity, i]], ssem[slot])

    pltpu.sync_copy(counts_hbm.at[pl.ds(tid * 16, 16)], cnt_v)
    nit = cnt_v[pl.ds(0, 16)][0]

    @pl.when(nit > 0)
    def _():
        idx_start(0, 0)

    pltpu.sync_copy(zeros_hbm, rows_v.at[pl.ds(0, _ZR)])
    pltpu.sync_copy(rows_v.at[pl.ds(0, _ZR)], acc.at[pl.ds(sid * _ZR, _ZR)])
    plsc.subcore_barrier()

    def body(b, carry):
        pb = b % 2
        idx_wait(b, pb)

        @pl.when(b > 0)
        def _():
            for slot in range(_RING):
                scatter(slot + _KI - _RING, 1 - pb, slot).wait()

        @pl.when(b < nit - 1)
        def _():
            idx_start(b + 1, 1 - pb)

        for i in range(_KI):
            slot = i % _RING
            if i >= _RING:
                scatter(i - _RING, pb, slot).wait()
            gather(i, pb, slot).start()
            if i >= 1:
                pslot = (i - 1) % _RING
                gather(i - 1, pb, pslot).wait()
                scatter(i - 1, pb, pslot).start(add=True)
        gather(_KI - 1, pb, (_KI - 1) % _RING).wait()
        scatter(_KI - 1, pb, (_KI - 1) % _RING).start(add=True)
        return carry

    lax.fori_loop(0, nit, body, 0)

    @pl.when(nit > 0)
    def _():
        pb_l = (nit - 1) % 2
        for slot in range(_RING):
            scatter(slot + _KI - _RING, pb_l, slot).wait()
    plsc.subcore_barrier()

    wb = sid * _ZR
    pltpu.sync_copy(acc.at[pl.ds(wb, _ZR)], rows_v.at[pl.ds(0, _ZR)])

    @pl.when(sid < 15)
    def _():
        pltpu.sync_copy(rows_v.at[pl.ds(0, _ZR)],
                        out_hbm.at[pl.ds(lo + wb, _ZR)])

    @pl.when(sid == 15)
    def _():
        pltpu.sync_copy(rows_v.at[pl.ds(0, 200)],
                        out_hbm.at[pl.ds(lo + wb, 200)])


@functools.lru_cache(maxsize=1)
def _stream_call():
    mesh = plsc.VectorSubcoreMesh(core_axis_name="c", subcore_axis_name="s")
    return pl.kernel(
        _stream_body,
        out_type=jax.ShapeDtypeStruct((N, DH), jnp.float32),
        mesh=mesh,
        scratch_types=[
            pltpu.VMEM((2, _KI, _GRP), jnp.int32),
            pltpu.VMEM((2, _KI, _GRP), jnp.int32),
            pltpu.VMEM((16,), jnp.int32),
            pltpu.VMEM((_RING * _GRP, DH), jnp.float32),
            pltpu.VMEM_SHARED((_NPAD, DH), jnp.float32),
        ] + [pltpu.SemaphoreType.DMA] * 10,
    )


def _mlp_body(h_ref, agg_ref, w1_ref, b1_ref, g_ref, be_ref, w2_ref,
              b2_ref, out_ref):
    h = h_ref[...] + agg_ref[...]
    y = lax.dot_general(h, w1_ref[...], (((1,), (0,)), ((), ())),
                        preferred_element_type=jnp.float32)
    y = y + b1_ref[...]
    m = jnp.mean(y, axis=0, keepdims=True)
    d = y - m
    v = jnp.mean(d * d, axis=0, keepdims=True)
    y = g_ref[...] * d * lax.rsqrt(v + BN_EPS) + be_ref[...]
    y = jnp.maximum(y, 0.0)
    y = lax.dot_general(y, w2_ref[...], (((1,), (0,)), ((), ())),
                        preferred_element_type=jnp.float32)
    y = y + b2_ref[...]
    out_ref[...] = jnp.maximum(y, 0.0)


def _mlp_call(h, agg, w1, b1, g, be, w2, b2):
    return pl.pallas_call(
        _mlp_body,
        out_shape=jax.ShapeDtypeStruct((N, DH), jnp.float32),
    )(h, agg, w1, b1.reshape(1, -1), g.reshape(1, -1), be.reshape(1, -1),
      w2, b2.reshape(1, -1))


def _tail_body(h_ref, agg_ref, w1_ref, b1_ref, g_ref, be_ref, w2_ref,
               b2_ref, batch_ref, lw1_ref, lb1_ref, lw2_ref, lb2_ref, out_ref):
    h = h_ref[...] + agg_ref[...]
    y = lax.dot_general(h, w1_ref[...], (((1,), (0,)), ((), ())),
                        preferred_element_type=jnp.float32)
    y = y + b1_ref[...]
    m = jnp.mean(y, axis=0, keepdims=True)
    d = y - m
    v = jnp.mean(d * d, axis=0, keepdims=True)
    y = g_ref[...] * d * lax.rsqrt(v + BN_EPS) + be_ref[...]
    y = jnp.maximum(y, 0.0)
    y = lax.dot_general(y, w2_ref[...], (((1,), (0,)), ((), ())),
                        preferred_element_type=jnp.float32)
    y = jnp.maximum(y + b2_ref[...], 0.0)
    seg_ids = batch_ref[...]
    cols = lax.broadcasted_iota(jnp.int32, (N, G), 1)
    onehot = jnp.where(cols == seg_ids, 1.0, 0.0).astype(jnp.float32)
    pooled = lax.dot_general(onehot, y, (((0,), (0,)), ((), ())),
                             preferred_element_type=jnp.float32)
    z = lax.dot_general(pooled, lw1_ref[...], (((1,), (0,)), ((), ())),
                        preferred_element_type=jnp.float32) + lb1_ref[...]
    z = jnp.maximum(z, 0.0)
    z = lax.dot_general(z, lw2_ref[...], (((1,), (0,)), ((), ())),
                        preferred_element_type=jnp.float32) + lb2_ref[...]
    out_ref[...] = 1.0 / (1.0 + jnp.exp(-z))


def _tail_call(h, agg, w1, b1, g, be, w2, b2, batch, lw1, lb1, lw2, lb2):
    return pl.pallas_call(
        _tail_body,
        out_shape=jax.ShapeDtypeStruct((G, 1), jnp.float32),
    )(h, agg, w1, b1.reshape(1, -1), g.reshape(1, -1), be.reshape(1, -1),
      w2, b2.reshape(1, -1), batch.reshape(N, 1), lw1,
      lb1.reshape(1, -1), lw2, lb2.reshape(1, -1))


def kernel(x, edge_index, batch, W11, b11, g1, be1, W12, b12, W21, b21, g2,
           be2, W22, b22, W31, b31, g3, be3, W32, b32, lw1, lb1, lw2, lb2):
    src = edge_index[0]
    dst = edge_index[1]
    pad = _EP - E
    src_p = jnp.concatenate([src, jnp.zeros((pad,), jnp.int32)])
    dst_p = jnp.concatenate([dst, jnp.full((pad,), N, jnp.int32)])
    own0 = dst_p < _HALF
    own1 = (dst_p >= _HALF) & (dst_p < N)
    ldst0 = jnp.where(own0, dst_p, _DUMMY)
    ldst1 = jnp.where(own1, dst_p - _HALF, _DUMMY)
    dstg = jnp.concatenate([ldst0, ldst1]).reshape(-1, _GRP)
    srcg = src_p.reshape(-1, _GRP)
    zeros = jnp.zeros((_ZR, DH), jnp.float32)

    csf, cdf, counts = _filter_call()(srcg, dstg)
    css = csf.reshape(-1, _GRP)
    cds = cdf.reshape(-1, _GRP)

    x_pad = jnp.pad(x, ((0, 0), (0, DH - DIN)))
    W11p = jnp.pad(W11, ((0, DH - DIN), (0, 0)))

    stream = _stream_call()
    agg = stream(x_pad, css, cds, counts, zeros)
    h = _mlp_call(x_pad, agg, W11p, b11, g1, be1, W12, b12)
    agg = stream(h, css, cds, counts, zeros)
    h = _mlp_call(h, agg, W21, b21, g2, be2, W22, b22)
    agg = stream(h, css, cds, counts, zeros)
    return _tail_call(h, agg, W31, b31, g3, be3, W32, b32,
                      batch, lw1, lb1, lw2, lb2)
